# baseline (speedup 1.0000x reference)
import functools

import jax
import jax.numpy as jnp
from jax import lax
from jax.experimental import pallas as pl
from jax.experimental.pallas import tpu as pltpu
from jax.experimental.pallas import tpu_sc as plsc

D_MODEL = 768
D_FF = 384
N_EXPERTS = 16
N_TOK = 2048

TILE = 128
G = N_TOK // TILE + N_EXPERTS
S = G * TILE

NC, NS = 2, 16
NW = NC * NS
CH = N_TOK // NW

BLK = 1024


def _router_body(x_ref, gw_ref, p_ref, w_ref, te_ref, gc_ref, oh_ref):
    x = x_ref[...]
    gw = gw_ref[...]
    logits = lax.dot_general(x, gw, (((1,), (1,)), ((), ())),
                             preferred_element_type=jnp.float32)
    m = jnp.max(logits, axis=1, keepdims=True)
    s = jnp.sum(jnp.exp(logits - m), axis=1, keepdims=True)
    w_ref[...] = jnp.broadcast_to(1.0 / s, (N_TOK, 128))

    eiota = lax.broadcasted_iota(jnp.int32, (N_TOK, N_EXPERTS), 1)
    eidx = jnp.min(jnp.where(logits == m, eiota, N_EXPERTS), axis=1,
                   keepdims=True)
    onehot = (eiota == eidx).astype(jnp.float32)
    oh_ref[...] = onehot

    counts = jnp.sum(onehot, axis=0, keepdims=True).astype(jnp.int32)
    pc = ((counts + (TILE - 1)) // TILE) * TILE
    pcf = pc.astype(jnp.float32)
    ii = lax.broadcasted_iota(jnp.int32, (N_EXPERTS, N_EXPERTS), 0)
    jj = lax.broadcasted_iota(jnp.int32, (N_EXPERTS, N_EXPERTS), 1)
    upper = (ii < jj).astype(jnp.float32)
    offs = lax.dot_general(pcf, upper, (((1,), (0,)), ((), ())),
                           preferred_element_type=jnp.float32,
                           precision=lax.Precision.HIGHEST)

    gt = lax.broadcasted_iota(jnp.int32, (G, N_EXPERTS), 0).astype(
        jnp.float32) * TILE
    ef = lax.broadcasted_iota(jnp.int32, (G, N_EXPERTS), 1).astype(jnp.float32)
    tmask = (gt >= offs) & (gt < offs + pcf)
    te_ref[...] = jnp.sum(jnp.where(tmask, ef, 0.0), axis=1,
                          keepdims=True).astype(jnp.int32)
    n_real = jnp.sum(pc) // TILE
    gi = lax.broadcasted_iota(jnp.int32, (G, 1), 0)
    gc_ref[...] = jnp.minimum(gi, n_real - 1)

    bi = lax.broadcasted_iota(jnp.int32, (BLK, BLK), 0)
    bj = lax.broadcasted_iota(jnp.int32, (BLK, BLK), 1)
    tril = (bj < bi).astype(jnp.float32)

    lane = lax.broadcasted_iota(jnp.int32, (BLK, 128), 1)
    rowm = lax.broadcasted_iota(jnp.int32, (BLK, 128), 0) % 128
    sel = (lane == rowm).astype(jnp.float32)
    rsel = (lax.broadcasted_iota(jnp.int32, (BLK // 128, BLK), 1) // 128 ==
            lax.broadcasted_iota(jnp.int32, (BLK // 128, BLK), 0)
            ).astype(jnp.float32)

    def body(b, carry):
        oh = oh_ref[pl.ds(b * BLK, BLK), :]
        ranks = lax.dot_general(tril, oh, (((1,), (0,)), ((), ())),
                                preferred_element_type=jnp.float32,
                                precision=lax.Precision.HIGHEST) + carry
        pos = jnp.sum(oh * (ranks + offs), axis=1, keepdims=True)
        posb = jnp.broadcast_to(pos, (BLK, 128)) * sel
        prows = lax.dot_general(rsel, posb, (((1,), (0,)), ((), ())),
                                preferred_element_type=jnp.float32,
                                precision=lax.Precision.HIGHEST)
        p_ref[pl.ds(b * (BLK // 128), BLK // 128), :] = prows.astype(jnp.int32)
        return carry + jnp.sum(oh, axis=0, keepdims=True)

    lax.fori_loop(0, N_TOK // BLK, body, jnp.zeros((1, N_EXPERTS), jnp.float32))


_router = pl.pallas_call(
    _router_body,
    out_shape=[
        jax.ShapeDtypeStruct((N_TOK // 128, 128), jnp.int32),
        jax.ShapeDtypeStruct((N_TOK, 128), jnp.float32),
        jax.ShapeDtypeStruct((G, 1), jnp.int32),
        jax.ShapeDtypeStruct((G, 1), jnp.int32),
    ],
    scratch_shapes=[pltpu.VMEM((N_TOK, N_EXPERTS), jnp.float32)],
)


def _expert_body(te_ref, gc_ref, xs_ref, ws_ref, fc1_ref, fc2_ref, ys_ref):
    i = pl.program_id(0)

    @pl.when(gc_ref[i] == i)
    def _():
        xb = xs_ref[...]
        h = lax.dot_general(xb, fc1_ref[0], (((1,), (1,)), ((), ())),
                            preferred_element_type=jnp.float32)
        h1 = h[:, :D_FF]
        g = h[:, D_FF:]
        act = h1 * (g / (1.0 + jnp.exp(-g)))
        y = lax.dot_general(act, fc2_ref[0], (((1,), (1,)), ((), ())),
                            preferred_element_type=jnp.float32)
        ys_ref[...] = y * ws_ref[:, 0:1]


_grouped = pl.pallas_call(
    _expert_body,
    grid_spec=pltpu.PrefetchScalarGridSpec(
        num_scalar_prefetch=2,
        grid=(G,),
        in_specs=[
            pl.BlockSpec((TILE, D_MODEL), lambda i, te, gc: (gc[i], 0)),
            pl.BlockSpec((TILE, 128), lambda i, te, gc: (gc[i], 0)),
            pl.BlockSpec((1, 2 * D_FF, D_MODEL),
                         lambda i, te, gc: (te[gc[i]], 0, 0)),
            pl.BlockSpec((1, D_MODEL, D_FF),
                         lambda i, te, gc: (te[gc[i]], 0, 0)),
        ],
        out_specs=pl.BlockSpec((TILE, D_MODEL), lambda i, te, gc: (gc[i], 0)),
    ),
    out_shape=jax.ShapeDtypeStruct((S, D_MODEL), jnp.float32),
)

def _sc_scatter_body(x_hbm, p_hbm, w_hbm, xs_hbm, ws_hbm,
                     idx_v, rows_v, w_v, sem):
    wid = lax.axis_index("s") * NC + lax.axis_index("c")
    base = wid * CH
    pltpu.sync_copy(p_hbm.at[pl.ds(base, CH)], idx_v)
    pltpu.sync_copy(x_hbm.at[pl.ds(base, CH)], rows_v)
    pltpu.sync_copy(w_hbm.at[pl.ds(base, CH)], w_v)
    cp = pltpu.async_copy(rows_v, xs_hbm.at[idx_v], sem)
    pltpu.async_copy(w_v, ws_hbm.at[idx_v], sem).wait()
    cp.wait()


def _sc_gather_body(ys_hbm, p_hbm, out_hbm, idx_v, rows_v, sem):
    wid = lax.axis_index("s") * NC + lax.axis_index("c")
    base = wid * CH
    pltpu.sync_copy(p_hbm.at[pl.ds(base, CH)], idx_v)
    pltpu.async_copy(ys_hbm.at[idx_v], rows_v, sem).wait()
    pltpu.sync_copy(rows_v, out_hbm.at[pl.ds(base, CH)])


@functools.lru_cache(maxsize=1)
def _sc_kernels():
    mesh = plsc.VectorSubcoreMesh(core_axis_name="c", subcore_axis_name="s",
                                  num_cores=NC, num_subcores=NS)
    scatter = pl.kernel(
        _sc_scatter_body,
        mesh=mesh,
        out_type=(
            jax.ShapeDtypeStruct((S, D_MODEL), jnp.float32),
            jax.ShapeDtypeStruct((S, 128), jnp.float32),
        ),
        scratch_types=[
            pltpu.VMEM((CH,), jnp.int32),
            pltpu.VMEM((CH, D_MODEL), jnp.float32),
            pltpu.VMEM((CH, 128), jnp.float32),
            pltpu.SemaphoreType.DMA,
        ],
    )
    gather = pl.kernel(
        _sc_gather_body,
        mesh=mesh,
        out_type=jax.ShapeDtypeStruct((N_TOK, D_MODEL), jnp.float32),
        scratch_types=[
            pltpu.VMEM((CH,), jnp.int32),
            pltpu.VMEM((CH, D_MODEL), jnp.float32),
            pltpu.SemaphoreType.DMA,
        ],
    )
    return scatter, gather


def kernel(x, gate_weight, fc1_weight, fc2_weight):
    p2, w2, te2, gc2 = _router(x, gate_weight)
    p = p2.reshape(N_TOK)
    te = te2.reshape(G)
    gc = gc2.reshape(G)
    sc_scatter, sc_gather = _sc_kernels()
    xs, ws = sc_scatter(x, p, w2)
    ys = _grouped(te, gc, xs, ws, fc1_weight, fc2_weight)
    return sc_gather(ys, p)


# speedup vs baseline: 1.0281x; 1.0281x over previous
import functools

import jax
import jax.numpy as jnp
from jax import lax
from jax.experimental import pallas as pl
from jax.experimental.pallas import tpu as pltpu
from jax.experimental.pallas import tpu_sc as plsc

D_MODEL = 768
D_FF = 384
N_EXPERTS = 16
N_TOK = 2048

TILE = 128
G = N_TOK // TILE + N_EXPERTS
S = G * TILE

NC, NS = 2, 16
NW = NC * NS
CH = N_TOK // NW

BLK = 256


def _router_body(x_ref, gw_ref, p_ref, w_ref, te_ref, gc_ref, oh_ref):
    x = x_ref[...]
    gw = gw_ref[...]
    logits = lax.dot_general(x, gw, (((1,), (1,)), ((), ())),
                             preferred_element_type=jnp.float32)
    m = jnp.max(logits, axis=1, keepdims=True)
    s = jnp.sum(jnp.exp(logits - m), axis=1, keepdims=True)
    w_ref[...] = jnp.broadcast_to(1.0 / s, (N_TOK, 128))

    eiota = lax.broadcasted_iota(jnp.int32, (N_TOK, N_EXPERTS), 1)
    eidx = jnp.min(jnp.where(logits == m, eiota, N_EXPERTS), axis=1,
                   keepdims=True)
    onehot = (eiota == eidx).astype(jnp.float32)
    oh_ref[...] = onehot

    counts = jnp.sum(onehot, axis=0, keepdims=True).astype(jnp.int32)
    pc = ((counts + (TILE - 1)) // TILE) * TILE
    pcf = pc.astype(jnp.float32)
    ii = lax.broadcasted_iota(jnp.int32, (N_EXPERTS, N_EXPERTS), 0)
    jj = lax.broadcasted_iota(jnp.int32, (N_EXPERTS, N_EXPERTS), 1)
    upper = (ii < jj).astype(jnp.float32)
    offs = lax.dot_general(pcf, upper, (((1,), (0,)), ((), ())),
                           preferred_element_type=jnp.float32,
                           precision=lax.Precision.HIGHEST)

    gt = lax.broadcasted_iota(jnp.int32, (G, N_EXPERTS), 0).astype(
        jnp.float32) * TILE
    ef = lax.broadcasted_iota(jnp.int32, (G, N_EXPERTS), 1).astype(jnp.float32)
    tmask = (gt >= offs) & (gt < offs + pcf)
    te_ref[...] = jnp.sum(jnp.where(tmask, ef, 0.0), axis=1,
                          keepdims=True).astype(jnp.int32)
    n_real = jnp.sum(pc) // TILE
    gi = lax.broadcasted_iota(jnp.int32, (G, 1), 0)
    gc_ref[...] = jnp.minimum(gi, n_real - 1)

    bi = lax.broadcasted_iota(jnp.int32, (BLK, BLK), 0)
    bj = lax.broadcasted_iota(jnp.int32, (BLK, BLK), 1)
    tril = (bj < bi).astype(jnp.float32)

    lane = lax.broadcasted_iota(jnp.int32, (BLK, 128), 1)
    rowm = lax.broadcasted_iota(jnp.int32, (BLK, 128), 0) % 128
    sel = (lane == rowm).astype(jnp.float32)
    rsel = (lax.broadcasted_iota(jnp.int32, (BLK // 128, BLK), 1) // 128 ==
            lax.broadcasted_iota(jnp.int32, (BLK // 128, BLK), 0)
            ).astype(jnp.float32)

    def body(b, carry):
        oh = oh_ref[pl.ds(b * BLK, BLK), :]
        ranks = lax.dot_general(tril, oh, (((1,), (0,)), ((), ())),
                                preferred_element_type=jnp.float32,
                                precision=lax.Precision.HIGHEST) + carry
        pos = jnp.sum(oh * (ranks + offs), axis=1, keepdims=True)
        posb = jnp.broadcast_to(pos, (BLK, 128)) * sel
        prows = lax.dot_general(rsel, posb, (((1,), (0,)), ((), ())),
                                preferred_element_type=jnp.float32,
                                precision=lax.Precision.HIGHEST)
        p_ref[pl.ds(b * (BLK // 128), BLK // 128), :] = prows.astype(jnp.int32)
        return carry + jnp.sum(oh, axis=0, keepdims=True)

    lax.fori_loop(0, N_TOK // BLK, body, jnp.zeros((1, N_EXPERTS), jnp.float32))


_router = pl.pallas_call(
    _router_body,
    out_shape=[
        jax.ShapeDtypeStruct((N_TOK // 128, 128), jnp.int32),
        jax.ShapeDtypeStruct((N_TOK, 128), jnp.float32),
        jax.ShapeDtypeStruct((G, 1), jnp.int32),
        jax.ShapeDtypeStruct((G, 1), jnp.int32),
    ],
    scratch_shapes=[pltpu.VMEM((N_TOK, N_EXPERTS), jnp.float32)],
)


def _expert_body(te_ref, gc_ref, xs_ref, ws_ref, fc1_ref, fc2_ref, ys_ref):
    i = pl.program_id(0)

    @pl.when(gc_ref[i] == i)
    def _():
        xb = xs_ref[...]
        h = lax.dot_general(xb, fc1_ref[0], (((1,), (1,)), ((), ())),
                            preferred_element_type=jnp.float32)
        h1 = h[:, :D_FF]
        g = h[:, D_FF:]
        act = h1 * (g / (1.0 + jnp.exp(-g)))
        y = lax.dot_general(act, fc2_ref[0], (((1,), (1,)), ((), ())),
                            preferred_element_type=jnp.float32)
        ys_ref[...] = y * ws_ref[:, 0:1]


_grouped = pl.pallas_call(
    _expert_body,
    grid_spec=pltpu.PrefetchScalarGridSpec(
        num_scalar_prefetch=2,
        grid=(G,),
        in_specs=[
            pl.BlockSpec((TILE, D_MODEL), lambda i, te, gc: (gc[i], 0)),
            pl.BlockSpec((TILE, 128), lambda i, te, gc: (gc[i], 0)),
            pl.BlockSpec((1, 2 * D_FF, D_MODEL),
                         lambda i, te, gc: (te[gc[i]], 0, 0)),
            pl.BlockSpec((1, D_MODEL, D_FF),
                         lambda i, te, gc: (te[gc[i]], 0, 0)),
        ],
        out_specs=pl.BlockSpec((TILE, D_MODEL), lambda i, te, gc: (gc[i], 0)),
    ),
    out_shape=jax.ShapeDtypeStruct((S, D_MODEL), jnp.float32),
)

def _sc_scatter_body(x_hbm, p_hbm, w_hbm, xs_hbm, ws_hbm,
                     idx_v, rows_v, w_v, sem):
    wid = lax.axis_index("s") * NC + lax.axis_index("c")
    base = wid * CH
    pltpu.sync_copy(p_hbm.at[pl.ds(base, CH)], idx_v)
    pltpu.sync_copy(x_hbm.at[pl.ds(base, CH)], rows_v)
    pltpu.sync_copy(w_hbm.at[pl.ds(base, CH)], w_v)
    cp = pltpu.async_copy(rows_v, xs_hbm.at[idx_v], sem)
    pltpu.async_copy(w_v, ws_hbm.at[idx_v], sem).wait()
    cp.wait()


def _sc_gather_body(ys_hbm, p_hbm, out_hbm, idx_v, rows_v, sem):
    wid = lax.axis_index("s") * NC + lax.axis_index("c")
    base = wid * CH
    pltpu.sync_copy(p_hbm.at[pl.ds(base, CH)], idx_v)
    pltpu.async_copy(ys_hbm.at[idx_v], rows_v, sem).wait()
    pltpu.sync_copy(rows_v, out_hbm.at[pl.ds(base, CH)])


@functools.lru_cache(maxsize=1)
def _sc_kernels():
    mesh = plsc.VectorSubcoreMesh(core_axis_name="c", subcore_axis_name="s",
                                  num_cores=NC, num_subcores=NS)
    scatter = pl.kernel(
        _sc_scatter_body,
        mesh=mesh,
        out_type=(
            jax.ShapeDtypeStruct((S, D_MODEL), jnp.float32),
            jax.ShapeDtypeStruct((S, 128), jnp.float32),
        ),
        scratch_types=[
            pltpu.VMEM((CH,), jnp.int32),
            pltpu.VMEM((CH, D_MODEL), jnp.float32),
            pltpu.VMEM((CH, 128), jnp.float32),
            pltpu.SemaphoreType.DMA,
        ],
    )
    gather = pl.kernel(
        _sc_gather_body,
        mesh=mesh,
        out_type=jax.ShapeDtypeStruct((N_TOK, D_MODEL), jnp.float32),
        scratch_types=[
            pltpu.VMEM((CH,), jnp.int32),
            pltpu.VMEM((CH, D_MODEL), jnp.float32),
            pltpu.SemaphoreType.DMA,
        ],
    )
    return scatter, gather


def kernel(x, gate_weight, fc1_weight, fc2_weight):
    p2, w2, te2, gc2 = _router(x, gate_weight)
    p = p2.reshape(N_TOK)
    te = te2.reshape(G)
    gc = gc2.reshape(G)
    sc_scatter, sc_gather = _sc_kernels()
    xs, ws = sc_scatter(x, p, w2)
    ys = _grouped(te, gc, xs, ws, fc1_weight, fc2_weight)
    return sc_gather(ys, p)


# speedup vs baseline: 1.0429x; 1.0145x over previous
import functools

import jax
import jax.numpy as jnp
from jax import lax
from jax.experimental import pallas as pl
from jax.experimental.pallas import tpu as pltpu
from jax.experimental.pallas import tpu_sc as plsc

D_MODEL = 768
D_FF = 384
N_EXPERTS = 16
N_TOK = 2048

TILE = 128
G = N_TOK // TILE + N_EXPERTS
S = G * TILE

NC, NS = 2, 16
NW = NC * NS
CH = N_TOK // NW

BLK = 256


def _router_body(x_ref, gw_ref, p_ref, w_ref, te_ref, gc_ref, oh_ref):
    x = x_ref[...]
    gw = gw_ref[...]
    logits = lax.dot_general(x, gw, (((1,), (1,)), ((), ())),
                             preferred_element_type=jnp.float32)
    m = jnp.max(logits, axis=1, keepdims=True)
    s = jnp.sum(jnp.exp(logits - m), axis=1, keepdims=True)
    w_ref[...] = jnp.broadcast_to(1.0 / s, (N_TOK, 128))

    eiota = lax.broadcasted_iota(jnp.int32, (N_TOK, N_EXPERTS), 1)
    eidx = jnp.min(jnp.where(logits == m, eiota, N_EXPERTS), axis=1,
                   keepdims=True)
    onehot = (eiota == eidx).astype(jnp.float32)
    oh_ref[...] = onehot

    counts = jnp.sum(onehot, axis=0, keepdims=True).astype(jnp.int32)
    pc = ((counts + (TILE - 1)) // TILE) * TILE
    pcf = pc.astype(jnp.float32)
    ii = lax.broadcasted_iota(jnp.int32, (N_EXPERTS, N_EXPERTS), 0)
    jj = lax.broadcasted_iota(jnp.int32, (N_EXPERTS, N_EXPERTS), 1)
    upper = (ii < jj).astype(jnp.float32)
    offs = lax.dot_general(pcf, upper, (((1,), (0,)), ((), ())),
                           preferred_element_type=jnp.float32)

    gt = lax.broadcasted_iota(jnp.int32, (G, N_EXPERTS), 0).astype(
        jnp.float32) * TILE
    ef = lax.broadcasted_iota(jnp.int32, (G, N_EXPERTS), 1).astype(jnp.float32)
    tmask = (gt >= offs) & (gt < offs + pcf)
    te_ref[...] = jnp.sum(jnp.where(tmask, ef, 0.0), axis=1,
                          keepdims=True).astype(jnp.int32)
    n_real = jnp.sum(pc) // TILE
    gi = lax.broadcasted_iota(jnp.int32, (G, 1), 0)
    gc_ref[...] = jnp.minimum(gi, n_real - 1)

    bi = lax.broadcasted_iota(jnp.int32, (BLK, BLK), 0)
    bj = lax.broadcasted_iota(jnp.int32, (BLK, BLK), 1)
    tril = (bj < bi).astype(jnp.float32)

    lane = lax.broadcasted_iota(jnp.int32, (BLK, 128), 1)
    rowm = lax.broadcasted_iota(jnp.int32, (BLK, 128), 0) % 128
    sel = (lane == rowm).astype(jnp.float32)
    rsel = (lax.broadcasted_iota(jnp.int32, (BLK // 128, BLK), 1) // 128 ==
            lax.broadcasted_iota(jnp.int32, (BLK // 128, BLK), 0)
            ).astype(jnp.float32)

    def body(b, carry):
        oh = oh_ref[pl.ds(b * BLK, BLK), :]
        ranks = lax.dot_general(tril, oh, (((1,), (0,)), ((), ())),
                                preferred_element_type=jnp.float32) + carry
        pos = jnp.sum(oh * (ranks + offs), axis=1, keepdims=True)
        posb = jnp.broadcast_to(pos, (BLK, 128)) * sel
        prows = lax.dot_general(rsel, posb, (((1,), (0,)), ((), ())),
                                preferred_element_type=jnp.float32,
                                precision=lax.Precision.HIGHEST)
        p_ref[pl.ds(b * (BLK // 128), BLK // 128), :] = prows.astype(jnp.int32)
        return carry + jnp.sum(oh, axis=0, keepdims=True)

    lax.fori_loop(0, N_TOK // BLK, body, jnp.zeros((1, N_EXPERTS), jnp.float32))


_router = pl.pallas_call(
    _router_body,
    out_shape=[
        jax.ShapeDtypeStruct((N_TOK // 128, 128), jnp.int32),
        jax.ShapeDtypeStruct((N_TOK, 128), jnp.float32),
        jax.ShapeDtypeStruct((G, 1), jnp.int32),
        jax.ShapeDtypeStruct((G, 1), jnp.int32),
    ],
    scratch_shapes=[pltpu.VMEM((N_TOK, N_EXPERTS), jnp.float32)],
)


def _expert_body(te_ref, gc_ref, xs_ref, ws_ref, fc1_ref, fc2_ref, ys_ref):
    i = pl.program_id(0)

    @pl.when(gc_ref[i] == i)
    def _():
        xb = xs_ref[...]
        h = lax.dot_general(xb, fc1_ref[0], (((1,), (1,)), ((), ())),
                            preferred_element_type=jnp.float32)
        h1 = h[:, :D_FF]
        g = h[:, D_FF:]
        act = h1 * (g / (1.0 + jnp.exp(-g)))
        y = lax.dot_general(act, fc2_ref[0], (((1,), (1,)), ((), ())),
                            preferred_element_type=jnp.float32)
        ys_ref[...] = y * ws_ref[:, 0:1]


_grouped = pl.pallas_call(
    _expert_body,
    grid_spec=pltpu.PrefetchScalarGridSpec(
        num_scalar_prefetch=2,
        grid=(G,),
        in_specs=[
            pl.BlockSpec((TILE, D_MODEL), lambda i, te, gc: (gc[i], 0)),
            pl.BlockSpec((TILE, 128), lambda i, te, gc: (gc[i], 0)),
            pl.BlockSpec((1, 2 * D_FF, D_MODEL),
                         lambda i, te, gc: (te[gc[i]], 0, 0)),
            pl.BlockSpec((1, D_MODEL, D_FF),
                         lambda i, te, gc: (te[gc[i]], 0, 0)),
        ],
        out_specs=pl.BlockSpec((TILE, D_MODEL), lambda i, te, gc: (gc[i], 0)),
    ),
    out_shape=jax.ShapeDtypeStruct((S, D_MODEL), jnp.float32),
)

def _sc_scatter_body(x_hbm, p_hbm, w_hbm, xs_hbm, ws_hbm,
                     idx_v, rows_v, w_v, sem):
    wid = lax.axis_index("s") * NC + lax.axis_index("c")
    base = wid * CH
    pltpu.sync_copy(p_hbm.at[pl.ds(base, CH)], idx_v)
    pltpu.sync_copy(x_hbm.at[pl.ds(base, CH)], rows_v)
    pltpu.sync_copy(w_hbm.at[pl.ds(base, CH)], w_v)
    cp = pltpu.async_copy(rows_v, xs_hbm.at[idx_v], sem)
    pltpu.async_copy(w_v, ws_hbm.at[idx_v], sem).wait()
    cp.wait()


def _sc_gather_body(ys_hbm, p_hbm, out_hbm, idx_v, rows_v, sem):
    wid = lax.axis_index("s") * NC + lax.axis_index("c")
    base = wid * CH
    pltpu.sync_copy(p_hbm.at[pl.ds(base, CH)], idx_v)
    pltpu.async_copy(ys_hbm.at[idx_v], rows_v, sem).wait()
    pltpu.sync_copy(rows_v, out_hbm.at[pl.ds(base, CH)])


@functools.lru_cache(maxsize=1)
def _sc_kernels():
    mesh = plsc.VectorSubcoreMesh(core_axis_name="c", subcore_axis_name="s",
                                  num_cores=NC, num_subcores=NS)
    scatter = pl.kernel(
        _sc_scatter_body,
        mesh=mesh,
        out_type=(
            jax.ShapeDtypeStruct((S, D_MODEL), jnp.float32),
            jax.ShapeDtypeStruct((S, 128), jnp.float32),
        ),
        scratch_types=[
            pltpu.VMEM((CH,), jnp.int32),
            pltpu.VMEM((CH, D_MODEL), jnp.float32),
            pltpu.VMEM((CH, 128), jnp.float32),
            pltpu.SemaphoreType.DMA,
        ],
    )
    gather = pl.kernel(
        _sc_gather_body,
        mesh=mesh,
        out_type=jax.ShapeDtypeStruct((N_TOK, D_MODEL), jnp.float32),
        scratch_types=[
            pltpu.VMEM((CH,), jnp.int32),
            pltpu.VMEM((CH, D_MODEL), jnp.float32),
            pltpu.SemaphoreType.DMA,
        ],
    )
    return scatter, gather


def kernel(x, gate_weight, fc1_weight, fc2_weight):
    p2, w2, te2, gc2 = _router(x, gate_weight)
    p = p2.reshape(N_TOK)
    te = te2.reshape(G)
    gc = gc2.reshape(G)
    sc_scatter, sc_gather = _sc_kernels()
    xs, ws = sc_scatter(x, p, w2)
    ys = _grouped(te, gc, xs, ws, fc1_weight, fc2_weight)
    return sc_gather(ys, p)
